# baseline (speedup 1.0000x reference)
import functools

import jax
import jax.numpy as jnp
from jax import lax
from jax.experimental import pallas as pl
from jax.experimental.pallas import tpu as pltpu
from jax.experimental.pallas import tpu_sc as plsc

_B = 16384
_D = 64
_NC = 2
_NS = 16
_NW = _NC * _NS
_BPW = _B // _NW
_CH = 128
_NCH = _BPW // _CH
_G = 16
_NG = _BPW // _G


def _sc_body(user_id, film_id, user_table, film_table, user_bias, film_bias,
             out, idx_u, idx_f, u_rows, f_rows, ub_v, fb_v, out_v, sem):
    wid = lax.axis_index("s") * _NC + lax.axis_index("c")
    base = wid * _BPW

    pltpu.sync_copy(user_id.at[pl.ds(base, _BPW)], idx_u)
    pltpu.sync_copy(film_id.at[pl.ds(base, _BPW)], idx_f)

    copies = []
    for j in range(_NCH):
        sl = pl.ds(j * _CH, _CH)
        copies.append(pltpu.async_copy(
            user_table.at[idx_u.at[sl]], u_rows.at[sl], sem))
        copies.append(pltpu.async_copy(
            film_table.at[idx_f.at[sl]], f_rows.at[sl], sem))
        copies.append(pltpu.async_copy(
            user_bias.at[idx_u.at[sl]], ub_v.at[sl], sem))
        copies.append(pltpu.async_copy(
            film_bias.at[idx_f.at[sl]], fb_v.at[sl], sem))
    for c in copies:
        c.wait()

    def group(g, carry):
        rows = g * _G + lax.iota(jnp.int32, _G)
        acc = jnp.zeros((_G,), jnp.float32)
        for d in range(_D):
            cols = jnp.full((_G,), d, jnp.int32)
            uu = plsc.load_gather(u_rows, [rows, cols])
            ff = plsc.load_gather(f_rows, [rows, cols])
            acc = acc + uu * ff
        sl16 = pl.ds(g * _G, _G)
        out_v[sl16] = acc + ub_v[sl16] + fb_v[sl16]
        return carry

    lax.fori_loop(0, _NG, group, 0)

    pltpu.sync_copy(out_v, out.at[pl.ds(base, _BPW)])


@jax.jit
def _run(user_id, film_id, user_table, film_table, user_bias, film_bias):
    mesh = plsc.VectorSubcoreMesh(core_axis_name="c", subcore_axis_name="s")
    f = pl.kernel(
        _sc_body,
        out_type=jax.ShapeDtypeStruct((_B,), jnp.float32),
        mesh=mesh,
        compiler_params=pltpu.CompilerParams(
            needs_layout_passes=False, use_tc_tiling_on_sc=False),
        scratch_types=[
            pltpu.VMEM((_BPW,), jnp.int32),
            pltpu.VMEM((_BPW,), jnp.int32),
            pltpu.VMEM((_BPW, _D), jnp.float32),
            pltpu.VMEM((_BPW, _D), jnp.float32),
            pltpu.VMEM((_BPW,), jnp.float32),
            pltpu.VMEM((_BPW,), jnp.float32),
            pltpu.VMEM((_BPW,), jnp.float32),
            pltpu.SemaphoreType.DMA,
        ],
    )
    return f(user_id, film_id, user_table, film_table, user_bias, film_bias)


def kernel(user_id, film_id, user_table, film_table, user_bias_table,
           film_bias_table):
    ub = user_bias_table.reshape((-1,))
    fb = film_bias_table.reshape((-1,))
    return _run(user_id, film_id, user_table, film_table, ub, fb)
